# Initial kernel scaffold; baseline (speedup 1.0000x reference)
import jax
import jax.numpy as jnp
from jax.experimental import pallas as pl


def kernel(x, edge_index, batch, params):
    raise NotImplementedError("write your pallas kernel here")


# speedup vs baseline: 1.0308x; 1.0308x over previous
import functools

import jax
import jax.numpy as jnp
from jax.experimental import pallas as pl
from jax.experimental.pallas import tpu as pltpu

N = 10000
NPAD = 10240
HID = 256
HEADS = 4
G = 16


def _qkvs_body(h_ref, wq_ref, bq_ref, wk_ref, bk_ref, wv_ref, bv_ref,
               ws_ref, bs_ref, q_ref, k_ref, v_ref, s_ref):
    h = h_ref[...]
    q_ref[...] = jnp.dot(h, wq_ref[...], preferred_element_type=jnp.float32) + bq_ref[...]
    k_ref[...] = jnp.dot(h, wk_ref[...], preferred_element_type=jnp.float32) + bk_ref[...]
    v_ref[...] = jnp.dot(h, wv_ref[...], preferred_element_type=jnp.float32) + bv_ref[...]
    s_ref[...] = jnp.dot(h, ws_ref[...], preferred_element_type=jnp.float32) + bs_ref[...]


def _qkvs(h, p):
    fi = h.shape[1]
    fo = HEADS * HID
    rb = 256
    grid = (NPAD // rb,)
    return pl.pallas_call(
        _qkvs_body,
        grid=grid,
        in_specs=[
            pl.BlockSpec((rb, fi), lambda r: (r, 0)),
            pl.BlockSpec((fi, fo), lambda r: (0, 0)),
            pl.BlockSpec((1, fo), lambda r: (0, 0)),
            pl.BlockSpec((fi, fo), lambda r: (0, 0)),
            pl.BlockSpec((1, fo), lambda r: (0, 0)),
            pl.BlockSpec((fi, fo), lambda r: (0, 0)),
            pl.BlockSpec((1, fo), lambda r: (0, 0)),
            pl.BlockSpec((fi, HID), lambda r: (0, 0)),
            pl.BlockSpec((1, HID), lambda r: (0, 0)),
        ],
        out_specs=[
            pl.BlockSpec((rb, fo), lambda r: (r, 0)),
            pl.BlockSpec((rb, fo), lambda r: (r, 0)),
            pl.BlockSpec((rb, fo), lambda r: (r, 0)),
            pl.BlockSpec((rb, HID), lambda r: (r, 0)),
        ],
        out_shape=[
            jax.ShapeDtypeStruct((NPAD, fo), jnp.float32),
            jax.ShapeDtypeStruct((NPAD, fo), jnp.float32),
            jax.ShapeDtypeStruct((NPAD, fo), jnp.float32),
            jax.ShapeDtypeStruct((NPAD, HID), jnp.float32),
        ],
    )(h, p["Wq"], p["bq"][None, :], p["Wk"], p["bk"][None, :],
      p["Wv"], p["bv"][None, :], p["Wskip"], p["bskip"][None, :])


def _attention_jnp(q, k, v, s, src, dst, last):
    q = q[:N].reshape(N, HEADS, HID)
    k = k[:N].reshape(N, HEADS, HID)
    v = v[:N].reshape(N, HEADS, HID)
    alpha = (q[dst] * k[src]).sum(-1) / jnp.sqrt(jnp.float32(HID))
    ex = jnp.exp(alpha)
    denom = jax.ops.segment_sum(ex, dst, num_segments=N)
    w = ex / (denom[dst] + 1e-16)
    msg = v[src] * w[:, :, None]
    out = jax.ops.segment_sum(msg, dst, num_segments=N).mean(axis=1)
    out = out + s[:N]
    out = jax.nn.relu(out)
    return out


def _ln_jnp(h, g, b, eps=1e-5):
    mu = h.mean(-1, keepdims=True)
    var = ((h - mu) ** 2).mean(-1, keepdims=True)
    return (h - mu) / jnp.sqrt(var + eps) * g + b


def kernel(x, edge_index, batch, params):
    src = edge_index[0]
    dst = edge_index[1]
    h = jnp.pad(x, ((0, NPAD - N), (0, 0)))
    for i in range(3):
        q, k, v, s = _qkvs(h, params["convs"][i])
        out = _attention_jnp(q, k, v, s, src, dst, i == 2)
        if i != 2:
            ln = params["lns"][i]
            out = _ln_jnp(out, ln["g"], ln["b"])
        h = jnp.pad(out, ((0, NPAD - N), (0, 0)))
    hN = h[:N]
    sums = jax.ops.segment_sum(hN, batch, num_segments=G)
    cnts = jax.ops.segment_sum(jnp.ones((N,), jnp.float32), batch, num_segments=G)
    pooled = sums / jnp.clip(cnts, 1.0)[:, None]
    return pooled @ params["post_W"] + params["post_b"]
